# baseline (speedup 1.0000x reference)
import functools

import jax
import jax.numpy as jnp
from jax import lax
from jax.experimental import pallas as pl
from jax.experimental.pallas import tpu as pltpu
from jax.experimental.pallas import tpu_sc as plsc

_N = 16384
_K = 128
_NUM_CORES = 2
_NUM_SUBCORES = 16
_NUM_WORKERS = _NUM_CORES * _NUM_SUBCORES
_CHUNK = _N // _NUM_WORKERS
_LANES = 16

_mesh = plsc.VectorSubcoreMesh(core_axis_name="c", subcore_axis_name="s")


@functools.partial(
    pl.kernel,
    out_type=jax.ShapeDtypeStruct((_N,), jnp.int32),
    mesh=_mesh,
    scratch_types=[
        pltpu.VMEM((_CHUNK,), jnp.int32),
        pltpu.VMEM((_K,), jnp.int32),
        pltpu.VMEM((_CHUNK,), jnp.int32),
    ],
    compiler_params=pltpu.CompilerParams(needs_layout_passes=False),
)
def _lookup(x_hbm, cond_hbm, out_hbm, x_v, cond_v, out_v):
    wid = lax.axis_index("s") * _NUM_CORES + lax.axis_index("c")
    base = wid * _CHUNK
    pltpu.sync_copy(x_hbm.at[pl.ds(base, _CHUNK)], x_v)
    pltpu.sync_copy(cond_hbm, cond_v)
    for i in range(_CHUNK // _LANES):
        query = x_v[pl.ds(i * _LANES, _LANES)]
        pos = jnp.zeros((_LANES,), jnp.int32)
        w = _K // 2
        while w >= 1:
            probe = plsc.load_gather(cond_v, [pos + (w - 1)])
            pos = jnp.where(probe < query, pos + w, pos)
            w //= 2
        out_v[pl.ds(i * _LANES, _LANES)] = pos
    pltpu.sync_copy(out_v, out_hbm.at[pl.ds(base, _CHUNK)])


def kernel(x, condition_tensors):
    idx = _lookup(x, condition_tensors)
    return idx.reshape(-1, 1, 1).astype(jnp.int64)


# speedup vs baseline: 1.1794x; 1.1794x over previous
import functools

import jax
import jax.numpy as jnp
from jax import lax
from jax.experimental import pallas as pl
from jax.experimental.pallas import tpu as pltpu
from jax.experimental.pallas import tpu_sc as plsc

_N = 16384
_K = 128
_NUM_CORES = 2
_NUM_SUBCORES = 16
_NUM_WORKERS = _NUM_CORES * _NUM_SUBCORES
_CHUNK = _N // _NUM_WORKERS
_LANES = 16

_mesh = plsc.VectorSubcoreMesh(core_axis_name="c", subcore_axis_name="s")


@functools.partial(
    pl.kernel,
    out_type=jax.ShapeDtypeStruct((_N,), jnp.int32),
    mesh=_mesh,
    scratch_types=[
        pltpu.VMEM((_CHUNK,), jnp.int32),
        pltpu.VMEM((_K,), jnp.int32),
        pltpu.VMEM((_K,), jnp.int32),
        pltpu.VMEM((_CHUNK,), jnp.int32),
    ],
    compiler_params=pltpu.CompilerParams(needs_layout_passes=False),
)
def _lookup(x_hbm, cond_hbm, out_hbm, x_v, cond_v, inv_v, out_v):
    wid = lax.axis_index("s") * _NUM_CORES + lax.axis_index("c")
    base = wid * _CHUNK
    pltpu.sync_copy(x_hbm.at[pl.ds(base, _CHUNK)], x_v)
    pltpu.sync_copy(cond_hbm, cond_v)
    lane = lax.iota(jnp.int32, _LANES)
    for k in range(_K // _LANES):
        vals = cond_v[pl.ds(k * _LANES, _LANES)]
        plsc.store_scatter(inv_v, [vals], lane + (k * _LANES))
    for i in range(_CHUNK // _LANES):
        query = x_v[pl.ds(i * _LANES, _LANES)]
        out_v[pl.ds(i * _LANES, _LANES)] = plsc.load_gather(inv_v, [query])
    pltpu.sync_copy(out_v, out_hbm.at[pl.ds(base, _CHUNK)])


def kernel(x, condition_tensors):
    idx = _lookup(x, condition_tensors)
    return idx.reshape(-1, 1, 1).astype(jnp.int64)


# speedup vs baseline: 1.2582x; 1.0669x over previous
import functools

import jax
import jax.numpy as jnp
from jax import lax
from jax.experimental import pallas as pl
from jax.experimental.pallas import tpu as pltpu
from jax.experimental.pallas import tpu_sc as plsc

_N = 16384
_K = 128
_NUM_CORES = 1
_NUM_SUBCORES = 16
_NUM_WORKERS = _NUM_CORES * _NUM_SUBCORES
_CHUNK = _N // _NUM_WORKERS
_LANES = 16

_mesh = plsc.VectorSubcoreMesh(core_axis_name="c", subcore_axis_name="s", num_cores=1)


@functools.partial(
    pl.kernel,
    out_type=jax.ShapeDtypeStruct((_N,), jnp.int32),
    mesh=_mesh,
    scratch_types=[
        pltpu.VMEM((_CHUNK,), jnp.int32),
        pltpu.VMEM((_K,), jnp.int32),
        pltpu.VMEM((_K,), jnp.int32),
        pltpu.VMEM((_CHUNK,), jnp.int32),
    ],
    compiler_params=pltpu.CompilerParams(needs_layout_passes=False),
)
def _lookup(x_hbm, cond_hbm, out_hbm, x_v, cond_v, inv_v, out_v):
    wid = lax.axis_index("s") * _NUM_CORES + lax.axis_index("c")
    base = wid * _CHUNK
    pltpu.sync_copy(x_hbm.at[pl.ds(base, _CHUNK)], x_v)
    pltpu.sync_copy(cond_hbm, cond_v)
    lane = lax.iota(jnp.int32, _LANES)
    for k in range(_K // _LANES):
        vals = cond_v[pl.ds(k * _LANES, _LANES)]
        plsc.store_scatter(inv_v, [vals], lane + (k * _LANES))
    for i in range(_CHUNK // _LANES):
        query = x_v[pl.ds(i * _LANES, _LANES)]
        out_v[pl.ds(i * _LANES, _LANES)] = plsc.load_gather(inv_v, [query])
    pltpu.sync_copy(out_v, out_hbm.at[pl.ds(base, _CHUNK)])


def kernel(x, condition_tensors):
    idx = _lookup(x, condition_tensors)
    return idx.reshape(-1, 1, 1).astype(jnp.int64)


# speedup vs baseline: 1.2832x; 1.0198x over previous
import functools
import jax
import jax.numpy as jnp
from jax import lax
from jax.experimental import pallas as pl
from jax.experimental.pallas import tpu as pltpu
from jax.experimental.pallas import tpu_sc as plsc

_N = 16384
_K = 128
_NUM_SUBCORES = 16
_CHUNK = _N // _NUM_SUBCORES
_LANES = 16

_mesh = plsc.VectorSubcoreMesh(core_axis_name="c", subcore_axis_name="s", num_cores=1)


@functools.partial(
    pl.kernel,
    out_type=jax.ShapeDtypeStruct((_N,), jnp.int32),
    mesh=_mesh,
    scratch_types=[
        pltpu.VMEM((_CHUNK,), jnp.int32),
        pltpu.VMEM((_K,), jnp.int32),
        pltpu.VMEM((_K,), jnp.int32),
        pltpu.VMEM((_CHUNK,), jnp.int32),
        pltpu.SemaphoreType.DMA,
        pltpu.SemaphoreType.DMA,
    ],
    compiler_params=pltpu.CompilerParams(
        needs_layout_passes=False,
        skip_device_barrier=True,
        disable_bounds_checks=True,
        disable_semaphore_checks=True,
    ),
)
def _lookup(x_hbm, cond_hbm, out_hbm, x_v, cond_v, inv_v, out_v, sem_x, sem_c):
    wid = lax.axis_index("s")
    base = wid * _CHUNK
    cp_x = pltpu.async_copy(x_hbm.at[pl.ds(base, _CHUNK)], x_v, sem_x)
    cp_c = pltpu.async_copy(cond_hbm, cond_v, sem_c)
    cp_c.wait()
    lane = lax.iota(jnp.int32, _LANES)
    for k in range(_K // _LANES):
        vals = cond_v[pl.ds(k * _LANES, _LANES)]
        plsc.store_scatter(inv_v, [vals], lane + (k * _LANES))
    cp_x.wait()
    for i in range(_CHUNK // _LANES):
        query = x_v[pl.ds(i * _LANES, _LANES)]
        out_v[pl.ds(i * _LANES, _LANES)] = plsc.load_gather(inv_v, [query])
    pltpu.sync_copy(out_v, out_hbm.at[pl.ds(base, _CHUNK)])


def kernel(x, condition_tensors):
    idx = _lookup(x, condition_tensors)
    return idx.reshape(-1, 1, 1).astype(jnp.int64)


# speedup vs baseline: 1.3205x; 1.0290x over previous
import functools
import jax
import jax.numpy as jnp
from jax import lax
from jax.experimental import pallas as pl
from jax.experimental.pallas import tpu as pltpu
from jax.experimental.pallas import tpu_sc as plsc

_N = 16384
_K = 128
_NUM_SUBCORES = 16
_CHUNK = _N // _NUM_SUBCORES
_LANES = 16

_mesh = plsc.VectorSubcoreMesh(core_axis_name="c", subcore_axis_name="s", num_cores=1)


@functools.partial(
    pl.kernel,
    out_type=jax.ShapeDtypeStruct((_N,), jnp.int32),
    mesh=_mesh,
    scratch_types=[
        pltpu.VMEM((_CHUNK,), jnp.int32),
        pltpu.VMEM((_K,), jnp.int32),
        pltpu.VMEM((_K,), jnp.int32),
        pltpu.VMEM((_CHUNK,), jnp.int32),
        pltpu.SemaphoreType.DMA,
        pltpu.SemaphoreType.DMA,
    ],
    compiler_params=pltpu.CompilerParams(
        needs_layout_passes=False,
        skip_device_barrier=True,
        disable_bounds_checks=True,
        disable_semaphore_checks=True,
    ),
)
def _lookup(x_hbm, cond_hbm, out_hbm, x_v, cond_v, inv_v, out_v, sem_x, sem_c):
    wid = lax.axis_index("s")
    base = wid * _CHUNK
    cp_x = pltpu.async_copy(x_hbm.at[pl.ds(base, _CHUNK)], x_v, sem_x)
    cp_c = pltpu.async_copy(cond_hbm, cond_v, sem_c)
    cp_c.wait()
    lane = lax.iota(jnp.int32, _LANES)
    for k in range(_K // _LANES):
        vals = cond_v[pl.ds(k * _LANES, _LANES)]
        plsc.store_scatter(inv_v, [vals], lane + (k * _LANES))
    cp_x.wait()
    def body(i, carry):
        off = i * _LANES
        query = x_v[pl.ds(off, _LANES)]
        out_v[pl.ds(off, _LANES)] = plsc.load_gather(inv_v, [query])
        return carry
    lax.fori_loop(0, _CHUNK // _LANES, body, 0)
    pltpu.sync_copy(out_v, out_hbm.at[pl.ds(base, _CHUNK)])


def kernel(x, condition_tensors):
    idx = _lookup(x, condition_tensors)
    return idx.reshape(-1, 1, 1).astype(jnp.int64)


# speedup vs baseline: 1.3322x; 1.0089x over previous
import functools
import jax
import jax.numpy as jnp
from jax import lax
from jax.experimental import pallas as pl
from jax.experimental.pallas import tpu as pltpu
from jax.experimental.pallas import tpu_sc as plsc

_N = 16384
_K = 128
_NUM_SUBCORES = 16
_CHUNK = _N // _NUM_SUBCORES
_LANES = 16

_mesh = plsc.VectorSubcoreMesh(core_axis_name="c", subcore_axis_name="s", num_cores=1)


@functools.partial(
    pl.kernel,
    out_type=jax.ShapeDtypeStruct((_N,), jnp.int32),
    mesh=_mesh,
    scratch_types=[
        pltpu.VMEM((_CHUNK,), jnp.int32),
        pltpu.VMEM((_K,), jnp.int32),
        pltpu.VMEM((_K,), jnp.int32),
        pltpu.VMEM((_CHUNK,), jnp.int32),
        pltpu.SemaphoreType.DMA,
        pltpu.SemaphoreType.DMA,
    ],
    compiler_params=pltpu.CompilerParams(
        needs_layout_passes=False,
        skip_device_barrier=True,
        disable_bounds_checks=True,
        disable_semaphore_checks=True,
    ),
)
def _lookup(x_hbm, cond_hbm, out_hbm, x_v, cond_v, inv_v, out_v, sem_x, sem_c):
    wid = lax.axis_index("s")
    base = wid * _CHUNK
    cp_x = pltpu.async_copy(x_hbm.at[pl.ds(base, _CHUNK)], x_v, sem_x)
    cp_c = pltpu.async_copy(cond_hbm, cond_v, sem_c)
    cp_c.wait()
    lane = lax.iota(jnp.int32, _LANES)
    for k in range(_K // _LANES):
        vals = cond_v[pl.ds(k * _LANES, _LANES)]
        plsc.store_scatter(inv_v, [vals], lane + (k * _LANES))
    cp_x.wait()
    @plsc.parallel_loop(0, _CHUNK, step=_LANES, unroll=4)
    def body(off):
        query = x_v[pl.ds(off, _LANES)]
        out_v[pl.ds(off, _LANES)] = plsc.load_gather(inv_v, [query])
    pltpu.sync_copy(out_v, out_hbm.at[pl.ds(base, _CHUNK)])


def kernel(x, condition_tensors):
    idx = _lookup(x, condition_tensors)
    return idx.reshape(-1, 1, 1).astype(jnp.int64)


# speedup vs baseline: 1.3334x; 1.0009x over previous
import functools
import jax
import jax.numpy as jnp
from jax import lax
from jax.experimental import pallas as pl
from jax.experimental.pallas import tpu as pltpu
from jax.experimental.pallas import tpu_sc as plsc

_N = 16384
_K = 128
_NUM_SUBCORES = 16
_CHUNK = _N // _NUM_SUBCORES
_LANES = 16

_mesh = plsc.VectorSubcoreMesh(core_axis_name="c", subcore_axis_name="s", num_cores=1)


@functools.partial(
    pl.kernel,
    out_type=jax.ShapeDtypeStruct((_N,), jnp.int32),
    mesh=_mesh,
    scratch_types=[
        pltpu.VMEM((_CHUNK,), jnp.int32),
        pltpu.VMEM((_K,), jnp.int32),
        pltpu.VMEM((_K,), jnp.int32),
        pltpu.VMEM((_CHUNK,), jnp.int32),
        pltpu.SemaphoreType.DMA,
        pltpu.SemaphoreType.DMA,
    ],
    compiler_params=pltpu.CompilerParams(
        needs_layout_passes=False,
        skip_device_barrier=True,
        disable_bounds_checks=True,
        disable_semaphore_checks=True,
    ),
)
def _lookup(x_hbm, cond_hbm, out_hbm, x_v, cond_v, inv_v, out_v, sem_x, sem_c):
    wid = lax.axis_index("s")
    base = wid * _CHUNK
    cp_x = pltpu.async_copy(x_hbm.at[pl.ds(base, _CHUNK)], x_v, sem_x)
    cp_c = pltpu.async_copy(cond_hbm, cond_v, sem_c)
    cp_c.wait()
    lane = lax.iota(jnp.int32, _LANES)
    @plsc.parallel_loop(0, _K, step=_LANES)
    def build(off):
        vals = cond_v[pl.ds(off, _LANES)]
        plsc.store_scatter(inv_v, [vals], lane + off)
    cp_x.wait()
    @plsc.parallel_loop(0, _CHUNK, step=_LANES, unroll=2)
    def body(off):
        query = x_v[pl.ds(off, _LANES)]
        out_v[pl.ds(off, _LANES)] = plsc.load_gather(inv_v, [query])
    pltpu.sync_copy(out_v, out_hbm.at[pl.ds(base, _CHUNK)])


def kernel(x, condition_tensors):
    idx = _lookup(x, condition_tensors)
    return idx.reshape(-1, 1, 1).astype(jnp.int64)


# speedup vs baseline: 1.3349x; 1.0012x over previous
import functools
import jax
import jax.numpy as jnp
from jax import lax
from jax.experimental import pallas as pl
from jax.experimental.pallas import tpu as pltpu
from jax.experimental.pallas import tpu_sc as plsc

_N = 16384
_K = 128
_NUM_SUBCORES = 16
_CHUNK = _N // _NUM_SUBCORES
_LANES = 16

_mesh = plsc.VectorSubcoreMesh(core_axis_name="c", subcore_axis_name="s", num_cores=1)


@functools.partial(
    pl.kernel,
    out_type=jax.ShapeDtypeStruct((_N,), jnp.int32),
    mesh=_mesh,
    scratch_types=[
        pltpu.VMEM((_CHUNK,), jnp.int32),
        pltpu.VMEM((_K,), jnp.int32),
        pltpu.VMEM((_K,), jnp.int32),
        pltpu.VMEM((_CHUNK,), jnp.int32),
        pltpu.SemaphoreType.DMA,
        pltpu.SemaphoreType.DMA,
    ],
    compiler_params=pltpu.CompilerParams(
        needs_layout_passes=False,
        use_tc_tiling_on_sc=False,
        skip_device_barrier=True,
        disable_bounds_checks=True,
        disable_semaphore_checks=True,
    ),
)
def _lookup(x_hbm, cond_hbm, out_hbm, x_v, cond_v, inv_v, out_v, sem_x, sem_c):
    wid = lax.axis_index("s")
    base = wid * _CHUNK
    cp_x = pltpu.async_copy(x_hbm.at[pl.ds(base, _CHUNK)], x_v, sem_x)
    cp_c = pltpu.async_copy(cond_hbm, cond_v, sem_c)
    cp_c.wait()
    lane = lax.iota(jnp.int32, _LANES)
    @plsc.parallel_loop(0, _K, step=_LANES)
    def build(off):
        vals = cond_v[pl.ds(off, _LANES)]
        plsc.store_scatter(inv_v, [vals], lane + off)
    cp_x.wait()
    @plsc.parallel_loop(0, _CHUNK, step=_LANES, unroll=2)
    def body(off):
        query = x_v[pl.ds(off, _LANES)]
        out_v[pl.ds(off, _LANES)] = plsc.load_gather(inv_v, [query])
    pltpu.sync_copy(out_v, out_hbm.at[pl.ds(base, _CHUNK)])


def kernel(x, condition_tensors):
    idx = _lookup(x, condition_tensors)
    return idx.reshape(-1, 1, 1).astype(jnp.int64)
